# baseline (speedup 1.0000x reference)
import jax
import jax.numpy as jnp
from jax.experimental import pallas as pl
from jax.experimental.pallas import tpu as pltpu

_BATCH_BLOCK = 64


def _dma_body(pos_emb_hbm, out_hbm, tab, buf, sem_in, sems):
    nchunk = out_hbm.shape[0] // _BATCH_BLOCK
    cp = pltpu.make_async_copy(pos_emb_hbm, tab, sem_in)
    cp.start()
    cp.wait()
    buf[...] = jnp.broadcast_to(tab[...][None], buf.shape)
    for c in range(nchunk):
        pltpu.make_async_copy(
            buf, out_hbm.at[pl.ds(c * _BATCH_BLOCK, _BATCH_BLOCK)], sems.at[c]
        ).start()
    for c in range(nchunk):
        pltpu.make_async_copy(
            buf, out_hbm.at[pl.ds(c * _BATCH_BLOCK, _BATCH_BLOCK)], sems.at[c]
        ).wait()


def kernel(x, pos_emb):
    batch = x.shape[0]
    seq, dim = pos_emb.shape
    nchunk = batch // _BATCH_BLOCK
    return pl.pallas_call(
        _dma_body,
        in_specs=[pl.BlockSpec(memory_space=pl.ANY)],
        out_specs=pl.BlockSpec(memory_space=pl.ANY),
        out_shape=jax.ShapeDtypeStruct((batch, seq, dim), jnp.float32),
        scratch_shapes=[
            pltpu.VMEM((seq, dim), jnp.float32),
            pltpu.VMEM((_BATCH_BLOCK, seq, dim), jnp.float32),
            pltpu.SemaphoreType.DMA,
            pltpu.SemaphoreType.DMA((nchunk,)),
        ],
    )(pos_emb)
